# initial kernel scaffold (unmeasured)
import jax
import jax.numpy as jnp
from jax import lax
from jax.experimental import pallas as pl
from jax.experimental.pallas import tpu as pltpu

M = 4096
NC = 1024
R = 256
MAXC = M // R
TAIL = MAXC
NSEM = MAXC + 1


def kernel(x, dest):
    p = lax.axis_index("x")
    order = jnp.argsort(dest, stable=True)
    xs = x.astype(jnp.bfloat16)[order]

    L = jnp.sum(dest == (1 - p)).astype(jnp.int32)
    S = jnp.where(p == 0, M - L, 0).astype(jnp.int32)
    D = jnp.where(p == 0, 0, M - L).astype(jnp.int32)
    K = jnp.where(p == 0, 0, L).astype(jnp.int32)
    scal = jnp.stack([L, S, D, K])

    def body(s_ref, x_ref, out_ref, send_sems, recv_sems, copy_sems):
        my_x = lax.axis_index("x")
        my_y = lax.axis_index("y")
        peer = (1 - my_x, my_y)

        L = s_ref[0]
        S = s_ref[1]
        D = s_ref[2]
        K = s_ref[3]
        keep_len = M - L
        n_full = L // R
        rem = L - n_full * R
        k_full = keep_len // R
        k_rem = keep_len - k_full * R

        barrier_sem = pltpu.get_barrier_semaphore()
        pl.semaphore_signal(
            barrier_sem, inc=1, device_id=peer,
            device_id_type=pl.DeviceIdType.MESH,
        )
        pl.semaphore_wait(barrier_sem, 1)

        def swap_desc(src_off, dst_off, sem_i):
            return pltpu.make_async_remote_copy(
                src_ref=x_ref.at[pl.ds(src_off, R), :],
                dst_ref=out_ref.at[pl.ds(dst_off, R), :],
                send_sem=send_sems.at[sem_i],
                recv_sem=recv_sems.at[sem_i],
                device_id=peer,
                device_id_type=pl.DeviceIdType.MESH,
            )

        def keep_desc(off, sem_i):
            return pltpu.make_async_copy(
                x_ref.at[pl.ds(off, R), :],
                out_ref.at[pl.ds(off, R), :],
                copy_sems.at[sem_i],
            )

        for i in range(MAXC):
            @pl.when(i < n_full)
            def _():
                swap_desc(S + i * R, D + i * R, i).start()

        @pl.when(rem > 0)
        def _():
            swap_desc(S + L - R, D + L - R, TAIL).start()

        for i in range(MAXC):
            @pl.when(i < k_full)
            def _():
                keep_desc(K + i * R, i).start()

        @pl.when(k_rem > 0)
        def _():
            keep_desc(K + keep_len - R, TAIL).start()

        for i in range(MAXC):
            @pl.when(i < k_full)
            def _():
                keep_desc(K + i * R, i).wait()

        @pl.when(k_rem > 0)
        def _():
            keep_desc(K + keep_len - R, TAIL).wait()

        for i in range(MAXC):
            @pl.when(i < n_full)
            def _():
                swap_desc(S + i * R, S + i * R, i).wait()

        @pl.when(rem > 0)
        def _():
            swap_desc(S + L - R, S + L - R, TAIL).wait()

    return pl.pallas_call(
        body,
        out_shape=jax.ShapeDtypeStruct((M, NC), jnp.bfloat16),
        in_specs=[
            pl.BlockSpec(memory_space=pltpu.SMEM),
            pl.BlockSpec(memory_space=pltpu.VMEM),
        ],
        out_specs=pl.BlockSpec(memory_space=pltpu.VMEM),
        scratch_shapes=[
            pltpu.SemaphoreType.DMA((NSEM,)),
            pltpu.SemaphoreType.DMA((NSEM,)),
            pltpu.SemaphoreType.DMA((NSEM,)),
        ],
        compiler_params=pltpu.CompilerParams(collective_id=0),
    )(scal, xs)


# baseline (device time: 118135 ns/iter reference)
import jax
import jax.numpy as jnp
from jax import lax
from jax.experimental import pallas as pl
from jax.experimental.pallas import tpu as pltpu

M = 4096
NC = 1024
G = 8
NGRP = M // G
CH = 32
MAXC = NGRP // CH
TAIL = MAXC
PKT = MAXC + 1


def kernel(x, dest):
    p = lax.axis_index("x")
    order = jnp.argsort(dest, stable=True)
    xs = x.astype(jnp.bfloat16)[order]

    L = jnp.sum(dest == (1 - p)).astype(jnp.int32)
    D = jnp.where(p == 0, 0, M - L)
    S = jnp.where(p == 0, M - L, 0)
    send_buf = jnp.roll(xs, D % G - S, axis=0)

    F = (D + G - 1) // G
    E8 = (D + L) // G
    NG = E8 - F
    A = F - D // G
    BP = jnp.where(p == 0, E8, 0)
    b = jnp.where(p == 0, S // G, (S + L) // G)
    lo = S - G * b
    hi = S + L - G * b
    KF = jnp.where(p == 0, 0, (L + G - 1) // G)
    KE = jnp.where(p == 0, (M - L) // G, NGRP)
    l8 = L % G
    scal = jnp.stack([NG, F, A, BP, b, lo, hi, KF, KE - KF, l8]).astype(
        jnp.int32
    )

    xs3 = xs.reshape(NGRP, G, NC)
    sb3 = send_buf.reshape(NGRP, G, NC)

    def body(s_ref, x_ref, sb_ref, out_ref, bnd_ref,
             send_sems, recv_sems, copy_sems):
        my_x = lax.axis_index("x")
        my_y = lax.axis_index("y")
        peer = (1 - my_x, my_y)

        NG = s_ref[0]
        F = s_ref[1]
        A = s_ref[2]
        BP = s_ref[3]
        b = s_ref[4]
        lo = s_ref[5]
        hi = s_ref[6]
        KF = s_ref[7]
        NK = s_ref[8]
        l8 = s_ref[9]

        barrier_sem = pltpu.get_barrier_semaphore()
        pl.semaphore_signal(
            barrier_sem, inc=1, device_id=peer,
            device_id_type=pl.DeviceIdType.MESH,
        )
        pl.semaphore_wait(barrier_sem, 1)

        def swap_desc(src_g, dst_g, sem_i):
            return pltpu.make_async_remote_copy(
                src_ref=sb_ref.at[pl.ds(src_g, CH)],
                dst_ref=out_ref.at[pl.ds(dst_g, CH)],
                send_sem=send_sems.at[sem_i],
                recv_sem=recv_sems.at[sem_i],
                device_id=peer,
                device_id_type=pl.DeviceIdType.MESH,
            )

        def pkt_desc():
            return pltpu.make_async_remote_copy(
                src_ref=sb_ref.at[pl.ds(BP, 1)],
                dst_ref=bnd_ref,
                send_sem=send_sems.at[PKT],
                recv_sem=recv_sems.at[PKT],
                device_id=peer,
                device_id_type=pl.DeviceIdType.MESH,
            )

        def keep_desc(g, sem_i):
            return pltpu.make_async_copy(
                x_ref.at[pl.ds(g, CH)],
                out_ref.at[pl.ds(g, CH)],
                copy_sems.at[sem_i],
            )

        @pl.when(l8 != 0)
        def _():
            pkt_desc().start()

        for k in range(MAXC):
            @pl.when(k * CH + CH <= NG)
            def _():
                swap_desc(A + k * CH, F + k * CH, k).start()

        @pl.when((NG % CH != 0) & (NG >= CH))
        def _():
            swap_desc(A + NG - CH, F + NG - CH, TAIL).start()

        for k in range(MAXC):
            @pl.when(k * CH + CH <= NK)
            def _():
                keep_desc(KF + k * CH, k).start()

        @pl.when((NK % CH != 0) & (NK >= CH))
        def _():
            keep_desc(KF + NK - CH, TAIL).start()

        @pl.when(l8 != 0)
        def _():
            pkt_desc().wait()
            ii = lax.broadcasted_iota(jnp.int32, (G, NC), 0)
            mask = (ii >= lo) & (ii < hi)
            out_ref[b] = jnp.where(mask, bnd_ref[0], x_ref[b])

        for k in range(MAXC):
            @pl.when(k * CH + CH <= NK)
            def _():
                keep_desc(KF + k * CH, k).wait()

        @pl.when((NK % CH != 0) & (NK >= CH))
        def _():
            keep_desc(KF + NK - CH, TAIL).wait()

        for k in range(MAXC):
            @pl.when(k * CH + CH <= NG)
            def _():
                swap_desc(A + k * CH, F + k * CH, k).wait()

        @pl.when((NG % CH != 0) & (NG >= CH))
        def _():
            swap_desc(A + NG - CH, F + NG - CH, TAIL).wait()

    out = pl.pallas_call(
        body,
        out_shape=jax.ShapeDtypeStruct((NGRP, G, NC), jnp.bfloat16),
        in_specs=[
            pl.BlockSpec(memory_space=pltpu.SMEM),
            pl.BlockSpec(memory_space=pltpu.VMEM),
            pl.BlockSpec(memory_space=pltpu.VMEM),
        ],
        out_specs=pl.BlockSpec(memory_space=pltpu.VMEM),
        scratch_shapes=[
            pltpu.VMEM((1, G, NC), jnp.bfloat16),
            pltpu.SemaphoreType.DMA((PKT + 1,)),
            pltpu.SemaphoreType.DMA((PKT + 1,)),
            pltpu.SemaphoreType.DMA((TAIL + 1,)),
        ],
        compiler_params=pltpu.CompilerParams(collective_id=0),
    )(scal, xs3, sb3)
    return out.reshape(M, NC)


# device time: 91404 ns/iter; 1.2924x vs baseline; 1.2924x over previous
import jax
import jax.numpy as jnp
from jax import lax
from jax.experimental import pallas as pl
from jax.experimental.pallas import tpu as pltpu

M = 4096
NC = 1024
SL = 8
LN = NC // SL
R = 256
MAXC = M // R
TAIL = MAXC
NSEM = MAXC + 1


def kernel(x, dest):
    p = lax.axis_index("x")
    order = jnp.argsort(dest, stable=True)
    xs = x.astype(jnp.bfloat16)[order]

    L = jnp.sum(dest == (1 - p)).astype(jnp.int32)
    S = jnp.where(p == 0, M - L, 0)
    D = jnp.where(p == 0, 0, M - L)
    K = jnp.where(p == 0, 0, L)
    scal = jnp.stack([L, S, D, K]).astype(jnp.int32)

    xs3 = xs.reshape(M, SL, LN)

    def body(s_ref, x_ref, out_ref, send_sems, recv_sems, copy_sems):
        my_x = lax.axis_index("x")
        my_y = lax.axis_index("y")
        peer = (1 - my_x, my_y)

        L = s_ref[0]
        S = s_ref[1]
        D = s_ref[2]
        K = s_ref[3]
        keep_len = M - L
        n_full = L // R
        rem = L - n_full * R
        k_full = keep_len // R
        k_rem = keep_len - k_full * R

        barrier_sem = pltpu.get_barrier_semaphore()
        pl.semaphore_signal(
            barrier_sem, inc=1, device_id=peer,
            device_id_type=pl.DeviceIdType.MESH,
        )
        pl.semaphore_wait(barrier_sem, 1)

        def swap_desc(src_off, dst_off, sem_i):
            return pltpu.make_async_remote_copy(
                src_ref=x_ref.at[pl.ds(src_off, R)],
                dst_ref=out_ref.at[pl.ds(dst_off, R)],
                send_sem=send_sems.at[sem_i],
                recv_sem=recv_sems.at[sem_i],
                device_id=peer,
                device_id_type=pl.DeviceIdType.MESH,
            )

        def keep_desc(off, sem_i):
            return pltpu.make_async_copy(
                x_ref.at[pl.ds(off, R)],
                out_ref.at[pl.ds(off, R)],
                copy_sems.at[sem_i],
            )

        for i in range(MAXC):
            @pl.when(i < n_full)
            def _():
                swap_desc(S + i * R, D + i * R, i).start()

        @pl.when(rem > 0)
        def _():
            swap_desc(S + L - R, D + L - R, TAIL).start()

        for i in range(MAXC):
            @pl.when(i < k_full)
            def _():
                keep_desc(K + i * R, i).start()

        @pl.when(k_rem > 0)
        def _():
            keep_desc(K + keep_len - R, TAIL).start()

        for i in range(MAXC):
            @pl.when(i < k_full)
            def _():
                keep_desc(K + i * R, i).wait()

        @pl.when(k_rem > 0)
        def _():
            keep_desc(K + keep_len - R, TAIL).wait()

        for i in range(MAXC):
            @pl.when(i < n_full)
            def _():
                swap_desc(S + i * R, S + i * R, i).wait()

        @pl.when(rem > 0)
        def _():
            swap_desc(S + L - R, S + L - R, TAIL).wait()

    out = pl.pallas_call(
        body,
        out_shape=jax.ShapeDtypeStruct((M, SL, LN), jnp.bfloat16),
        in_specs=[
            pl.BlockSpec(memory_space=pltpu.SMEM),
            pl.BlockSpec(memory_space=pltpu.VMEM),
        ],
        out_specs=pl.BlockSpec(memory_space=pltpu.VMEM),
        scratch_shapes=[
            pltpu.SemaphoreType.DMA((NSEM,)),
            pltpu.SemaphoreType.DMA((NSEM,)),
            pltpu.SemaphoreType.DMA((NSEM,)),
        ],
        compiler_params=pltpu.CompilerParams(collective_id=0),
    )(scal, xs3)
    return out.reshape(M, NC)


# device time: 88957 ns/iter; 1.3280x vs baseline; 1.0275x over previous
import jax
import jax.numpy as jnp
from jax import lax
from jax.experimental import pallas as pl
from jax.experimental.pallas import tpu as pltpu

M = 4096
NC = 1024
SL = 8
LN = NC // SL
R = 256
MAXC = M // R
TAIL = MAXC
NSEM = MAXC + 1


def kernel(x, dest):
    p = lax.axis_index("x")
    order = jnp.argsort(dest, stable=True)
    xs = x[order].astype(jnp.bfloat16)

    L = jnp.sum(dest == (1 - p)).astype(jnp.int32)
    S = jnp.where(p == 0, M - L, 0)
    D = jnp.where(p == 0, 0, M - L)
    K = jnp.where(p == 0, 0, L)
    scal = jnp.stack([L, S, D, K]).astype(jnp.int32)

    xs3 = xs.reshape(M, SL, LN)

    def body(s_ref, x_ref, out_ref, send_sems, recv_sems, copy_sems):
        my_x = lax.axis_index("x")
        my_y = lax.axis_index("y")
        peer = (1 - my_x, my_y)

        L = s_ref[0]
        S = s_ref[1]
        D = s_ref[2]
        K = s_ref[3]
        keep_len = M - L
        n_full = L // R
        rem = L - n_full * R
        k_full = keep_len // R
        k_rem = keep_len - k_full * R

        barrier_sem = pltpu.get_barrier_semaphore()
        pl.semaphore_signal(
            barrier_sem, inc=1, device_id=peer,
            device_id_type=pl.DeviceIdType.MESH,
        )
        pl.semaphore_wait(barrier_sem, 1)

        def swap_desc(src_off, dst_off, sem_i):
            return pltpu.make_async_remote_copy(
                src_ref=x_ref.at[pl.ds(src_off, R)],
                dst_ref=out_ref.at[pl.ds(dst_off, R)],
                send_sem=send_sems.at[sem_i],
                recv_sem=recv_sems.at[sem_i],
                device_id=peer,
                device_id_type=pl.DeviceIdType.MESH,
            )

        def keep_desc(off, sem_i):
            return pltpu.make_async_copy(
                x_ref.at[pl.ds(off, R)],
                out_ref.at[pl.ds(off, R)],
                copy_sems.at[sem_i],
            )

        for i in range(MAXC):
            @pl.when(i < n_full)
            def _():
                swap_desc(S + i * R, D + i * R, i).start()

        @pl.when(rem > 0)
        def _():
            swap_desc(S + L - R, D + L - R, TAIL).start()

        for i in range(MAXC):
            @pl.when(i < k_full)
            def _():
                keep_desc(K + i * R, i).start()

        @pl.when(k_rem > 0)
        def _():
            keep_desc(K + keep_len - R, TAIL).start()

        for i in range(MAXC):
            @pl.when(i < k_full)
            def _():
                keep_desc(K + i * R, i).wait()

        @pl.when(k_rem > 0)
        def _():
            keep_desc(K + keep_len - R, TAIL).wait()

        for i in range(MAXC):
            @pl.when(i < n_full)
            def _():
                swap_desc(S + i * R, S + i * R, i).wait()

        @pl.when(rem > 0)
        def _():
            swap_desc(S + L - R, S + L - R, TAIL).wait()

    out = pl.pallas_call(
        body,
        out_shape=jax.ShapeDtypeStruct((M, SL, LN), jnp.bfloat16),
        in_specs=[
            pl.BlockSpec(memory_space=pltpu.SMEM),
            pl.BlockSpec(memory_space=pltpu.MemorySpace.HBM),
        ],
        out_specs=pl.BlockSpec(memory_space=pltpu.MemorySpace.HBM),
        scratch_shapes=[
            pltpu.SemaphoreType.DMA((NSEM,)),
            pltpu.SemaphoreType.DMA((NSEM,)),
            pltpu.SemaphoreType.DMA((NSEM,)),
        ],
        compiler_params=pltpu.CompilerParams(collective_id=0),
    )(scal, xs3)
    return out.reshape(M, NC)
